# baseline (speedup 1.0000x reference)
import functools

import jax
import jax.numpy as jnp
from jax.experimental import pallas as pl
from jax.experimental.pallas import tpu as pltpu

EPS = 1e-09


def _logmse_step(y_pred_hbm, y_true_ref, out_ref, buf, sems, *, rows, nblocks,
                 inv_n):
    i = pl.program_id(0)
    slot = jax.lax.rem(i, 2)

    def chunk_copy(j, sl):
        return pltpu.make_async_copy(
            y_pred_hbm.at[pl.ds(j * rows, rows), :],
            buf.at[sl],
            sems.at[sl],
        )

    @pl.when(i == 0)
    def _first():
        chunk_copy(0, 0).start()

    @pl.when(i + 1 < nblocks)
    def _prefetch():
        chunk_copy(i + 1, 1 - slot).start()

    chunk_copy(i, slot).wait()
    yp = buf[slot]
    yt = y_true_ref[...]
    lp = jnp.log2(jnp.maximum(yp, EPS))
    lt = jnp.log2(jnp.maximum(yt, EPS))
    d = lt - lp
    partial = jnp.sum(d * d).reshape(1, 1)

    @pl.when(i == 0)
    def _init():
        out_ref[...] = partial

    @pl.when(i > 0)
    def _acc():
        out_ref[...] = out_ref[...] + partial

    @pl.when(i == nblocks - 1)
    def _finish():
        out_ref[...] = out_ref[...] * inv_n


def kernel(y_pred, y_true):
    b, s, _ = y_pred.shape
    lanes = 128
    reps = s // lanes
    n = b * reps
    yp = y_pred.reshape(n, lanes)
    yt = jnp.broadcast_to(y_true.reshape(b, 1, 1), (b, reps, 1)).reshape(n, 1)
    rows = 4096
    nblocks = n // rows
    inv_n = 1.0 / float(b * s)
    out = pl.pallas_call(
        functools.partial(_logmse_step, rows=rows, nblocks=nblocks,
                          inv_n=inv_n),
        grid=(nblocks,),
        in_specs=[
            pl.BlockSpec(memory_space=pltpu.MemorySpace.HBM),
            pl.BlockSpec((rows, 1), lambda i: (i, 0)),
        ],
        out_specs=pl.BlockSpec((1, 1), lambda i: (0, 0)),
        out_shape=jax.ShapeDtypeStruct((1, 1), jnp.float32),
        scratch_shapes=[
            pltpu.VMEM((2, rows, lanes), jnp.float32),
            pltpu.SemaphoreType.DMA((2,)),
        ],
    )(yp, yt)
    loss = out[0, 0]
    return (loss, loss, jnp.array(0.0, dtype=jnp.float32))
